# baseline (speedup 1.0000x reference)
import functools

import jax
import jax.numpy as jnp
from jax import lax
from jax.experimental import pallas as pl
from jax.experimental.pallas import tpu as pltpu
from jax.experimental.pallas import tpu_sc as plsc

_TOP_K = 8
_BLOCK_N = 1024
_E = 64
_L = 16
_UNROLL = 8


def _logits_block_kernel(x_ref, w_ref, b_ref, out_ref):
    dims = (((1,), (1,)), ((), ()))
    logits = jax.lax.dot_general(
        w_ref[...], x_ref[...], dims, preferred_element_type=jnp.float32)
    out_ref[...] = logits + b_ref[...].reshape(w_ref.shape[0], 1)


def _tc_logits(x, w_gate, b_gate):
    n, d = x.shape
    e = w_gate.shape[0]
    bn = min(_BLOCK_N, n)
    nb = n // bn
    cores = 2 if nb % 2 == 0 else 1
    half = nb // cores
    b2 = b_gate.reshape(1, e)
    return pl.pallas_call(
        _logits_block_kernel,
        grid=(cores, half),
        in_specs=[
            pl.BlockSpec((bn, d), lambda i, j: (i * half + j, 0)),
            pl.BlockSpec((e, d), lambda i, j: (0, 0)),
            pl.BlockSpec((1, e), lambda i, j: (0, 0)),
        ],
        out_specs=pl.BlockSpec((e, bn), lambda i, j: (0, i * half + j)),
        out_shape=jax.ShapeDtypeStruct((e, n), jnp.float32),
        compiler_params=pltpu.CompilerParams(
            dimension_semantics=("parallel", "arbitrary")),
    )(x, w_gate, b2)


def _make_sc_topk(n, e, k_top):
    info = plsc.get_sparse_core_info()
    nc, ns = info.num_cores, info.num_subcores
    nw = nc * ns
    tpw = n // nw
    ngrp = tpw // _L
    esteps = e // _UNROLL
    mesh = plsc.VectorSubcoreMesh(core_axis_name="c", subcore_axis_name="s")
    neg_inf = jnp.float32(-jnp.inf)

    @functools.partial(
        pl.kernel, mesh=mesh,
        out_type=[
            jax.ShapeDtypeStruct((e, n), jnp.float32),
            jax.ShapeDtypeStruct((nw, e, _L), jnp.int32),
        ],
        scratch_types=[
            pltpu.VMEM((e, tpw), jnp.float32),
            pltpu.VMEM((e, _L), jnp.float32),
            pltpu.VMEM((e, _L), jnp.float32),
            pltpu.VMEM((e, tpw), jnp.float32),
            pltpu.VMEM((e, _L), jnp.int32),
        ],
    )
    def sc_topk(logits_hbm, gates_hbm, loadp_hbm, lv, wv, gv, ov, cnt):
        wid = lax.axis_index("s") * nc + lax.axis_index("c")
        base = wid * tpw
        pltpu.sync_copy(logits_hbm.at[:, pl.ds(base, tpw)], lv)

        zero_i = jnp.zeros((_L,), jnp.int32)
        for ee in range(e):
            cnt[ee, :] = zero_i

        def group_body(g, _):
            col = g * _L

            def init_body(i, m0):
                for u in range(_UNROLL):
                    ee = i * _UNROLL + u
                    v = lv[ee, pl.ds(col, _L)]
                    wv[ee, :] = v
                    m0 = jnp.maximum(m0, v)
                return m0
            m0 = lax.fori_loop(0, esteps, init_body,
                               jnp.full((_L,), neg_inf, jnp.float32))

            def k_body(_, carry):
                def max_body(i, m):
                    for u in range(_UNROLL):
                        m = jnp.maximum(m, wv[i * _UNROLL + u, :])
                    return m
                m = lax.fori_loop(0, esteps, max_body,
                                  jnp.full((_L,), neg_inf, jnp.float32))

                def mask_body(i, found):
                    for u in range(_UNROLL):
                        ee = i * _UNROLL + u
                        v = wv[ee, :]
                        sel = jnp.where(v == m, 1 - found, 0)
                        found = found + sel
                        wv[ee, :] = jnp.where(sel > 0, neg_inf, v)
                    return found
                lax.fori_loop(0, esteps, mask_body,
                              jnp.zeros((_L,), jnp.int32))
                return carry
            lax.fori_loop(0, k_top, k_body, 0)

            def exp_body(i, denom):
                for u in range(_UNROLL):
                    ee = i * _UNROLL + u
                    v = lv[ee, pl.ds(col, _L)]
                    ex = jnp.where(wv[ee, :] == neg_inf,
                                   jnp.exp(v - m0), jnp.float32(0.0))
                    gv[ee, :] = ex
                    denom = denom + ex
                return denom
            denom = lax.fori_loop(0, esteps, exp_body,
                                  jnp.zeros((_L,), jnp.float32))

            def norm_body(i, _):
                for u in range(_UNROLL):
                    ee = i * _UNROLL + u
                    gval = gv[ee, :] / denom
                    ov[ee, pl.ds(col, _L)] = gval
                    cnt[ee, :] = cnt[ee, :] + jnp.where(
                        gval > 0, jnp.int32(1), jnp.int32(0))
                return 0
            lax.fori_loop(0, esteps, norm_body, 0)
            return 0

        lax.fori_loop(0, ngrp, group_body, 0)

        pltpu.sync_copy(ov, gates_hbm.at[:, pl.ds(base, tpw)])
        pltpu.sync_copy(cnt, loadp_hbm.at[wid])

    return sc_topk


def kernel(x, w_gate, b_gate, w_noise, b_noise):
    del w_noise, b_noise
    n, _ = x.shape
    e = w_gate.shape[0]
    logits_t = _tc_logits(x, w_gate, b_gate)
    gates_t, loadp = _make_sc_topk(n, e, _TOP_K)(logits_t)
    return gates_t.T, loadp.sum(axis=(0, 2))


# speedup vs baseline: 2.1202x; 2.1202x over previous
import functools

import jax
import jax.numpy as jnp
from jax.experimental import pallas as pl
from jax.experimental.pallas import tpu as pltpu

_TOP_K = 8
_BLOCK_N = 1024


def _gating_block_kernel(x_ref, w_ref, b_ref, gates_ref, load_ref, *, k_top):
    x = x_ref[...]
    w = w_ref[...]
    e = w.shape[0]
    bn = x.shape[0]
    logits = jax.lax.dot_general(
        w, x, (((1,), (1,)), ((), ())), preferred_element_type=jnp.float32)
    logits = logits + b_ref[...].reshape(e, 1)

    row = jax.lax.broadcasted_iota(jnp.int32, (e, bn), 0)
    work = logits
    m0 = jnp.max(work, axis=0, keepdims=True)
    for t in range(k_top):
        m = m0 if t == 0 else jnp.max(work, axis=0, keepdims=True)
        is_max = work == m
        sel = jnp.min(jnp.where(is_max, row, e), axis=0, keepdims=True)
        work = jnp.where(row == sel, -jnp.inf, work)
    selected = work == -jnp.inf
    ex = jnp.where(selected, jnp.exp(logits - m0), jnp.float32(0.0))
    denom = jnp.sum(ex, axis=0, keepdims=True)
    gates = (ex / denom).T
    gates_ref[...] = gates
    counts = jnp.sum((gates > 0).astype(jnp.int32), axis=0, keepdims=True)

    @pl.when(pl.program_id(1) == 0)
    def _init():
        load_ref[...] = counts[None]

    @pl.when(pl.program_id(1) != 0)
    def _accumulate():
        load_ref[...] += counts[None]


def kernel(x, w_gate, b_gate, w_noise, b_noise):
    del w_noise, b_noise
    n, d = x.shape
    e = w_gate.shape[0]
    bn = min(_BLOCK_N, n)
    nb = n // bn
    cores = 2 if nb % 2 == 0 else 1
    half = nb // cores
    b2 = b_gate.reshape(1, e)

    gates, load3 = pl.pallas_call(
        functools.partial(_gating_block_kernel, k_top=_TOP_K),
        grid=(cores, half),
        in_specs=[
            pl.BlockSpec((bn, d), lambda i, j: (i * half + j, 0)),
            pl.BlockSpec((e, d), lambda i, j: (0, 0)),
            pl.BlockSpec((1, e), lambda i, j: (0, 0)),
        ],
        out_specs=[
            pl.BlockSpec((bn, e), lambda i, j: (i * half + j, 0)),
            pl.BlockSpec((1, 1, e), lambda i, j: (i, 0, 0)),
        ],
        out_shape=[
            jax.ShapeDtypeStruct((n, e), x.dtype),
            jax.ShapeDtypeStruct((cores, 1, e), jnp.int32),
        ],
        compiler_params=pltpu.CompilerParams(
            dimension_semantics=("parallel", "arbitrary")),
    )(x, w_gate, b2)

    load = load3.sum(axis=(0, 1))
    return gates, load
